# Initial kernel scaffold; baseline (speedup 1.0000x reference)
import jax
import jax.numpy as jnp
from jax.experimental import pallas as pl


def kernel(rho, symbols, W, b):
    raise NotImplementedError("write your pallas kernel here")


# speedup vs baseline: 3.2289x; 3.2289x over previous
import jax
import jax.numpy as jnp
from jax import lax
from jax.experimental import pallas as pl
from jax.experimental.pallas import tpu as pltpu
from jax.experimental.pallas import tpu_sc as plsc

NTA = 16384
O = 2048
NMAX = 2048
E = 8

M = 256
P = NTA + E * M
NUM_BLOCKS = P // M

NC, NS = 2, 16
NW = NC * NS
ROWS_PER_W = NTA // NW
CHUNK = 32
NCHUNK = ROWS_PER_W // CHUNK

def _sc_mesh():
    return plsc.VectorSubcoreMesh(core_axis_name="c", subcore_axis_name="s")


def _worker_base():
    wid = lax.axis_index("s") * NC + lax.axis_index("c")
    return wid, wid * ROWS_PER_W


def _dispatch_body(rho_hbm, dest_hbm, xs_hbm, idx_v, buf, sem):
    wid, base = _worker_base()
    pltpu.sync_copy(dest_hbm.at[wid], idx_v)
    for j in range(NCHUNK):
        pltpu.sync_copy(rho_hbm.at[pl.ds(base + j * CHUNK, CHUNK)], buf)
        pltpu.async_copy(buf, xs_hbm.at[idx_v.at[j]], sem).wait()


def _combine_body(os_hbm, dest_hbm, out_hbm, idx_v, buf, sem):
    wid, base = _worker_base()
    pltpu.sync_copy(dest_hbm.at[wid], idx_v)
    for j in range(NCHUNK):
        pltpu.async_copy(os_hbm.at[idx_v.at[j]], buf, sem).wait()
        pltpu.sync_copy(buf, out_hbm.at[pl.ds(base + j * CHUNK, CHUNK)])


def _sc_dispatch(rho, dest3):
    return pl.kernel(
        _dispatch_body,
        out_type=jax.ShapeDtypeStruct((P, O), jnp.float32),
        mesh=_sc_mesh(),
        scratch_types=[
            pltpu.VMEM((NCHUNK, CHUNK), jnp.int32),
            pltpu.VMEM((CHUNK, O), jnp.float32),
            pltpu.SemaphoreType.DMA,
        ],
    )(rho, dest3)


def _sc_combine(out_sorted, dest3):
    return pl.kernel(
        _combine_body,
        out_type=jax.ShapeDtypeStruct((NTA, NMAX), jnp.float32),
        mesh=_sc_mesh(),
        scratch_types=[
            pltpu.VMEM((NCHUNK, CHUNK), jnp.int32),
            pltpu.VMEM((CHUNK, NMAX), jnp.float32),
            pltpu.SemaphoreType.DMA,
        ],
    )(out_sorted, dest3)


def _gmm_body(be_ref, x_ref, w_ref, b_ref, o_ref):
    del be_ref
    o_ref[...] = (
        jnp.dot(x_ref[...], w_ref[0], preferred_element_type=jnp.float32)
        + b_ref[0, 0]
    )


def _tc_grouped_matmul(block_expert, x_sorted, W, b):
    grid_spec = pltpu.PrefetchScalarGridSpec(
        num_scalar_prefetch=1,
        grid=(NUM_BLOCKS,),
        in_specs=[
            pl.BlockSpec((M, O), lambda i, be: (i, 0)),
            pl.BlockSpec((1, O, NMAX), lambda i, be: (be[i], 0, 0)),
            pl.BlockSpec((1, 1, NMAX), lambda i, be: (be[i], 0, 0)),
        ],
        out_specs=pl.BlockSpec((M, NMAX), lambda i, be: (i, 0)),
    )
    return pl.pallas_call(
        _gmm_body,
        grid_spec=grid_spec,
        out_shape=jax.ShapeDtypeStruct((P, NMAX), jnp.float32),
        compiler_params=pltpu.CompilerParams(
            dimension_semantics=("arbitrary",),
        ),
    )(block_expert, x_sorted, W, b.reshape(E, 1, NMAX))


def _routing(symbols):
    sym = symbols.astype(jnp.int32)
    onehot = sym[:, None] == jnp.arange(E, dtype=jnp.int32)[None, :]
    cum = jnp.cumsum(onehot.astype(jnp.int32), axis=0)
    counts = cum[-1]
    padded = ((counts + M - 1) // M) * M
    pad_end = jnp.cumsum(padded)
    pad_start = pad_end - padded
    dest = jnp.sum(
        jnp.where(onehot, cum - 1 + pad_start[None, :], 0), axis=1
    ).astype(jnp.int32)
    block_expert = jnp.minimum(
        jnp.searchsorted(
            pad_end, jnp.arange(NUM_BLOCKS, dtype=jnp.int32) * M, side="right"
        ),
        E - 1,
    ).astype(jnp.int32)
    return dest.reshape(NW, NCHUNK, CHUNK), block_expert


def kernel(rho, symbols, W, b):
    dest3, block_expert = _routing(symbols)
    x_sorted = _sc_dispatch(rho, dest3)
    out_sorted = _tc_grouped_matmul(block_expert, x_sorted, W, b)
    return _sc_combine(out_sorted, dest3)


# speedup vs baseline: 3.3185x; 1.0277x over previous
import jax
import jax.numpy as jnp
import numpy as np
from jax import lax
from jax.experimental import pallas as pl
from jax.experimental.pallas import tpu as pltpu
from jax.experimental.pallas import tpu_sc as plsc

NTA = 16384
O = 2048
NMAX = 2048
E = 8

M = 256
P = NTA + E * M
NUM_BLOCKS = P // M

NC, NS = 2, 16
NW = NC * NS
ROWS_PER_W = NTA // NW
CHUNK = 16
NCHUNK = ROWS_PER_W // CHUNK


def _sc_mesh():
    return plsc.VectorSubcoreMesh(core_axis_name="c", subcore_axis_name="s")


def _worker_base():
    wid = lax.axis_index("s") * NC + lax.axis_index("c")
    return wid, wid * ROWS_PER_W


def _dispatch_body(
    rho_hbm, dest_hbm, xs_hbm, idx_v, buf0, buf1, rsem0, rsem1, ssem0, ssem1
):
    wid, base = _worker_base()
    pltpu.sync_copy(dest_hbm.at[wid], idx_v)
    bufs = (buf0, buf1)
    rsems = (rsem0, rsem1)
    ssems = (ssem0, ssem1)

    def read(b, jj):
        return pltpu.make_async_copy(
            rho_hbm.at[pl.ds(base + jj * CHUNK, CHUNK)], bufs[b], rsems[b]
        )

    def scatter(b, jj):
        return pltpu.make_async_copy(bufs[b], xs_hbm.at[idx_v.at[jj]], ssems[b])

    read(0, 0).start()

    @pl.loop(0, NCHUNK, step=2)
    def _chunks(j):
        for b in range(2):
            jj = j + b
            nb = (b + 1) % 2

            @pl.when(jj + 1 < NCHUNK)
            def _next_read():
                @pl.when(jj >= 1)
                def _drain():
                    scatter(nb, jj - 1).wait()

                read(nb, jj + 1).start()

            read(b, jj).wait()
            scatter(b, jj).start()

    scatter(0, NCHUNK - 2).wait()
    scatter(1, NCHUNK - 1).wait()


def _combine_body(
    os_hbm, dest_hbm, out_hbm, idx_v, buf0, buf1, rsem0, rsem1, ssem0, ssem1
):
    wid, base = _worker_base()
    pltpu.sync_copy(dest_hbm.at[wid], idx_v)
    bufs = (buf0, buf1)
    rsems = (rsem0, rsem1)
    ssems = (ssem0, ssem1)

    def gather(b, jj):
        return pltpu.make_async_copy(os_hbm.at[idx_v.at[jj]], bufs[b], rsems[b])

    def writeback(b, jj):
        return pltpu.make_async_copy(
            bufs[b], out_hbm.at[pl.ds(base + jj * CHUNK, CHUNK)], ssems[b]
        )

    gather(0, 0).start()

    @pl.loop(0, NCHUNK, step=2)
    def _chunks(j):
        for b in range(2):
            jj = j + b
            nb = (b + 1) % 2

            @pl.when(jj + 1 < NCHUNK)
            def _next_gather():
                @pl.when(jj >= 1)
                def _drain():
                    writeback(nb, jj - 1).wait()

                gather(nb, jj + 1).start()

            gather(b, jj).wait()
            writeback(b, jj).start()

    writeback(0, NCHUNK - 2).wait()
    writeback(1, NCHUNK - 1).wait()


def _sc_dispatch(rho, dest3):
    return pl.kernel(
        _dispatch_body,
        out_type=jax.ShapeDtypeStruct((P, O), jnp.float32),
        mesh=_sc_mesh(),
        scratch_types=[
            pltpu.VMEM((NCHUNK, CHUNK), jnp.int32),
            pltpu.VMEM((CHUNK, O), jnp.float32),
            pltpu.VMEM((CHUNK, O), jnp.float32),
            pltpu.SemaphoreType.DMA,
            pltpu.SemaphoreType.DMA,
            pltpu.SemaphoreType.DMA,
            pltpu.SemaphoreType.DMA,
        ],
    )(rho, dest3)


def _sc_combine(out_sorted, dest3):
    return pl.kernel(
        _combine_body,
        out_type=jax.ShapeDtypeStruct((NTA, NMAX), jnp.float32),
        mesh=_sc_mesh(),
        scratch_types=[
            pltpu.VMEM((NCHUNK, CHUNK), jnp.int32),
            pltpu.VMEM((CHUNK, NMAX), jnp.float32),
            pltpu.VMEM((CHUNK, NMAX), jnp.float32),
            pltpu.SemaphoreType.DMA,
            pltpu.SemaphoreType.DMA,
            pltpu.SemaphoreType.DMA,
            pltpu.SemaphoreType.DMA,
        ],
    )(out_sorted, dest3)


def _gmm_body(be_ref, x_ref, w_ref, b_ref, o_ref, wbf_ref):
    i = pl.program_id(0)
    prev = be_ref[jnp.maximum(i - 1, 0)]

    @pl.when((i == 0) | (be_ref[i] != prev))
    def _convert():
        wbf_ref[...] = w_ref[0].astype(jnp.bfloat16)

    o_ref[...] = (
        jnp.dot(
            x_ref[...].astype(jnp.bfloat16),
            wbf_ref[...],
            preferred_element_type=jnp.float32,
        )
        + b_ref[0, 0]
    )


def _tc_grouped_matmul(block_expert, x_sorted, W, b):
    grid_spec = pltpu.PrefetchScalarGridSpec(
        num_scalar_prefetch=1,
        grid=(NUM_BLOCKS,),
        in_specs=[
            pl.BlockSpec((M, O), lambda i, be: (i, 0)),
            pl.BlockSpec((1, O, NMAX), lambda i, be: (be[i], 0, 0)),
            pl.BlockSpec((1, 1, NMAX), lambda i, be: (be[i], 0, 0)),
        ],
        out_specs=pl.BlockSpec((M, NMAX), lambda i, be: (i, 0)),
        scratch_shapes=[pltpu.VMEM((O, NMAX), jnp.bfloat16)],
    )
    return pl.pallas_call(
        _gmm_body,
        grid_spec=grid_spec,
        out_shape=jax.ShapeDtypeStruct((P, NMAX), jnp.float32),
        compiler_params=pltpu.CompilerParams(
            dimension_semantics=("arbitrary",),
        ),
    )(block_expert, x_sorted, W, b.reshape(E, 1, NMAX))


RH = 128
RL = NTA // RH
EF = E * RH


def _route_body(ps_ref, sym_ref, u_ref, b_ref, dest_ref):
    sym = sym_ref[...]
    e_ids = lax.broadcasted_iota(jnp.int32, (E, RH, RL), 0)
    oh = (sym[None] == e_ids).astype(jnp.float32).reshape(EF, RL)
    plo = jnp.dot(oh, u_ref[...], preferred_element_type=jnp.float32)
    tot = jnp.broadcast_to(plo[:, RL - 1 : RL], (EF, RL))
    rex = jnp.dot(b_ref[...], tot, preferred_element_type=jnp.float32)
    rank0 = (plo + rex - 1.0).astype(jnp.int32)
    blk = lax.broadcasted_iota(jnp.int32, (EF, RL), 0) // RH
    psb = jnp.zeros((EF, RL), jnp.int32)
    for e in range(E):
        psb = jnp.where(blk == e, ps_ref[e], psb)
    dflat = jnp.where(oh.astype(jnp.int32) > 0, rank0 + psb, 0)
    dest_ref[...] = jnp.sum(dflat.reshape(E, RH, RL), axis=0)


def _routing(symbols):
    sym = symbols.astype(jnp.int32)
    counts = jnp.sum(
        (sym[:, None] == jnp.arange(E, dtype=jnp.int32)[None, :]).astype(jnp.int32),
        axis=0,
    )
    padded = ((counts + M - 1) // M) * M
    pad_end = jnp.cumsum(padded)
    pad_start = (pad_end - padded).astype(jnp.int32)
    block_expert = jnp.minimum(
        jnp.searchsorted(
            pad_end, jnp.arange(NUM_BLOCKS, dtype=jnp.int32) * M, side="right"
        ),
        E - 1,
    ).astype(jnp.int32)
    r = np.arange(RL)
    u_mat = jnp.asarray((r[:, None] <= r[None, :]).astype(np.float32))
    i = np.arange(EF)
    b_mat = jnp.asarray(
        (
            (i[:, None] // RH == i[None, :] // RH)
            & (i[None, :] % RH < i[:, None] % RH)
        ).astype(np.float32)
    )
    dest = pl.pallas_call(
        _route_body,
        grid=(1,),
        in_specs=[
            pl.BlockSpec(memory_space=pltpu.SMEM),
            pl.BlockSpec((RH, RL), lambda i: (0, 0)),
            pl.BlockSpec((RL, RL), lambda i: (0, 0)),
            pl.BlockSpec((EF, EF), lambda i: (0, 0)),
        ],
        out_specs=pl.BlockSpec((RH, RL), lambda i: (0, 0)),
        out_shape=jax.ShapeDtypeStruct((RH, RL), jnp.int32),
    )(pad_start, sym.reshape(RH, RL), u_mat, b_mat)
    return dest.reshape(NW, NCHUNK, CHUNK), block_expert


def kernel(rho, symbols, W, b):
    dest3, block_expert = _routing(symbols)
    x_sorted = _sc_dispatch(rho, dest3)
    out_sorted = _tc_grouped_matmul(block_expert, x_sorted, W, b)
    return _sc_combine(out_sorted, dest3)
